# baseline (speedup 1.0000x reference)
import functools

import jax
import jax.numpy as jnp
from jax import lax
from jax.experimental import pallas as pl
from jax.experimental.pallas import tpu as pltpu
from jax.experimental.pallas import tpu_sc as plsc

NC = 2
NS = 16
NW = NC * NS
K = 128
NI = 3
NB = 2
UNROLL = 6


def _sc_agg_call(n_pad, spw, d):
    mesh = plsc.VectorSubcoreMesh(core_axis_name="c", subcore_axis_name="s")
    rows_per_tile = n_pad // NS

    @functools.partial(
        pl.kernel,
        mesh=mesh,
        out_type=jax.ShapeDtypeStruct((NC, n_pad, d), jnp.float32),
        scratch_types=(
            [pltpu.VMEM((K,), jnp.int32) for _ in range(2 * NI)]
            + [pltpu.VMEM((K, d), jnp.float32) for _ in range(NB)]
            + [pltpu.VMEM_SHARED((n_pad, d), jnp.float32)]
            + [pltpu.SemaphoreType.DMA for _ in range(NI + NB + 1)]
        ),
    )
    def sc_agg(xx_hbm, src_hbm, dst_hbm, out_hbm,
               src0, src1, src2, dst0, dst1, dst2, rows0, rows1,
               agg_sh, isem0, isem1, isem2, gsem0, gsem1, ssem):
        c = lax.axis_index("c")
        s = lax.axis_index("s")
        wid = c * NS + s
        srcs = (src0, src1, src2)
        dsts = (dst0, dst1, dst2)
        isems = (isem0, isem1, isem2)
        rows = (rows0, rows1)
        gsems = (gsem0, gsem1)
        rslc = pl.ds(s * rows_per_tile, rows_per_tile)
        base = wid * (spw * K)

        def idx_start(g, j):
            off = pl.multiple_of(base + g * K, K)
            pltpu.async_copy(src_hbm.at[pl.ds(off, K)], srcs[j], isems[j])
            pltpu.async_copy(dst_hbm.at[pl.ds(off, K)], dsts[j], isems[j])

        def idx_wait(g, j):
            off = pl.multiple_of(base + g * K, K)
            pltpu.make_async_copy(src_hbm.at[pl.ds(off, K)], srcs[j],
                                  isems[j]).wait()
            pltpu.make_async_copy(dst_hbm.at[pl.ds(off, K)], dsts[j],
                                  isems[j]).wait()

        def gather_start(j, b):
            pltpu.async_copy(xx_hbm.at[c].at[srcs[j]], rows[b], gsems[b])

        def gather_wait(j, b):
            pltpu.make_async_copy(xx_hbm.at[c].at[srcs[j]], rows[b],
                                  gsems[b]).wait()

        for g in range(NI):
            idx_start(g, g)
        pltpu.async_copy(xx_hbm.at[c, rslc], agg_sh.at[rslc], ssem)
        pltpu.make_async_copy(xx_hbm.at[c, rslc], agg_sh.at[rslc], ssem).wait()
        plsc.subcore_barrier()

        idx_wait(0, 0)
        gather_start(0, 0)
        idx_wait(1, 1)
        gather_start(1, 1)

        def step(i, carry):
            g0 = i * UNROLL
            for u in range(UNROLL):
                g = g0 + u
                b = u % NB
                j = u % NI
                jn = (u + 2) % NI
                gather_wait(j, b)
                pltpu.sync_copy(rows[b], agg_sh.at[dsts[j]], add=True)

                @pl.when(g + NI < spw)
                def _():
                    idx_start(g + NI, j)

                @pl.when(g + 2 < spw)
                def _():
                    idx_wait(g + 2, jn)
                    gather_start(jn, b)
            return carry

        lax.fori_loop(0, spw // UNROLL, step, 0)
        plsc.subcore_barrier()
        pltpu.sync_copy(agg_sh.at[rslc], out_hbm.at[c, rslc])

    return sc_agg


def _dense_body(n, xr, p0r, p1r, w1r, b1r, w2r, b2r, gr, br, outr):
    h = p0r[...][:n] + p1r[...][:n] - xr[...]
    a = jnp.dot(h, w1r[...], preferred_element_type=jnp.float32) + b1r[...]
    a = jnp.maximum(a, 0.0)
    h2 = jnp.dot(a, w2r[...], preferred_element_type=jnp.float32) + b2r[...]
    mean = jnp.mean(h2, axis=0, keepdims=True)
    cent = h2 - mean
    var = jnp.mean(cent * cent, axis=0, keepdims=True)
    scale = lax.rsqrt(var + 1e-5) * gr[...]
    outr[...] = jnp.maximum(cent * scale + br[...], 0.0)


def kernel(x, edge_index, W1, b1, W2, b2, gamma, beta):
    n, d = x.shape
    e = edge_index.shape[1]
    spw = -(-e // (NW * K))
    spw = -(-spw // UNROLL) * UNROLL
    e_pad = spw * K * NW
    n_pad = -(-(n + 1) // (NS * 8)) * (NS * 8)
    dummy = n_pad - 1

    src = edge_index[0].astype(jnp.int32)
    dst = edge_index[1].astype(jnp.int32)
    src_p = jnp.concatenate([src, jnp.zeros((e_pad - e,), jnp.int32)])
    dst_p = jnp.concatenate([dst, jnp.full((e_pad - e,), dummy, jnp.int32)])
    xp = jnp.pad(x, ((0, n_pad - n), (0, 0)))
    xx = jnp.stack([xp, xp])

    slabs = _sc_agg_call(n_pad, spw, d)(xx, src_p, dst_p)

    out = pl.pallas_call(
        functools.partial(_dense_body, n),
        out_shape=jax.ShapeDtypeStruct((n, d), jnp.float32),
    )(x, slabs[0], slabs[1], W1.T, b1.reshape(1, d), W2.T,
      b2.reshape(1, d), gamma.reshape(1, d), beta.reshape(1, d))
    return out


# speedup vs baseline: 3.4052x; 3.4052x over previous
import functools

import jax
import jax.numpy as jnp
from jax import lax
from jax.experimental import pallas as pl
from jax.experimental.pallas import tpu as pltpu
from jax.experimental.pallas import tpu_sc as plsc

NC = 2
NS = 16
K = 128
NI = 3
NB = 2
UNROLL = 6
SHARE0 = 0.87


def _sc_agg_call(n_pad, spw0, spw1, d):
    mesh = plsc.VectorSubcoreMesh(core_axis_name="c", subcore_axis_name="s")
    rows_per_tile = n_pad // NS

    @functools.partial(
        pl.kernel,
        mesh=mesh,
        out_type=jax.ShapeDtypeStruct((NC, n_pad, d), jnp.float32),
        scratch_types=(
            [pltpu.VMEM((K,), jnp.int32) for _ in range(2 * NI)]
            + [pltpu.VMEM((K, d), jnp.float32) for _ in range(NB)]
            + [pltpu.VMEM_SHARED((n_pad, d), jnp.float32)]
            + [pltpu.SemaphoreType.DMA for _ in range(NI + NB + 1)]
        ),
    )
    def sc_agg(xp_hbm, zz_hbm, src_hbm, dst_hbm, out_hbm,
               src0, src1, src2, dst0, dst1, dst2, rows0, rows1,
               agg_sh, isem0, isem1, isem2, gsem0, gsem1, ssem):
        c = lax.axis_index("c")
        s = lax.axis_index("s")
        srcs = (src0, src1, src2)
        dsts = (dst0, dst1, dst2)
        isems = (isem0, isem1, isem2)
        rows = (rows0, rows1)
        gsems = (gsem0, gsem1)
        rslc = pl.ds(s * rows_per_tile, rows_per_tile)
        base0 = s * (spw0 * K)
        base1 = (NS * spw0 + s * spw1) * K

        def idx_start(base, g, j):
            off = pl.multiple_of(base + g * K, K)
            pltpu.async_copy(src_hbm.at[pl.ds(off, K)], srcs[j], isems[j])
            pltpu.async_copy(dst_hbm.at[pl.ds(off, K)], dsts[j], isems[j])

        def idx_wait(base, g, j):
            off = pl.multiple_of(base + g * K, K)
            pltpu.make_async_copy(src_hbm.at[pl.ds(off, K)], srcs[j],
                                  isems[j]).wait()
            pltpu.make_async_copy(dst_hbm.at[pl.ds(off, K)], dsts[j],
                                  isems[j]).wait()

        def gather_start(j, b):
            pltpu.async_copy(xp_hbm.at[srcs[j]], rows[b], gsems[b])

        def gather_wait(j, b):
            pltpu.make_async_copy(xp_hbm.at[srcs[j]], rows[b],
                                  gsems[b]).wait()

        @pl.when(c == 0)
        def _():
            for g in range(NI):
                idx_start(base0, g, g)
            pltpu.async_copy(xp_hbm.at[rslc], agg_sh.at[rslc], ssem)
            pltpu.make_async_copy(xp_hbm.at[rslc], agg_sh.at[rslc],
                                  ssem).wait()

        @pl.when(c == 1)
        def _():
            for g in range(NI):
                idx_start(base1, g, g)
            pltpu.async_copy(zz_hbm.at[rslc], agg_sh.at[rslc], ssem)
            pltpu.make_async_copy(zz_hbm.at[rslc], agg_sh.at[rslc],
                                  ssem).wait()

        plsc.subcore_barrier()

        @pl.when(c == 0)
        def _():
            idx_wait(base0, 0, 0)
            gather_start(0, 0)
            idx_wait(base0, 1, 1)
            gather_start(1, 1)

            def step(i, carry):
                g0 = i * UNROLL
                for u in range(UNROLL):
                    g = g0 + u
                    b = u % NB
                    j = u % NI
                    jn = (u + 2) % NI
                    gather_wait(j, b)
                    pltpu.sync_copy(rows[b], agg_sh.at[dsts[j]], add=True)

                    @pl.when(g + NI < spw0)
                    def _():
                        idx_start(base0, g + NI, j)

                    @pl.when(g + 2 < spw0)
                    def _():
                        idx_wait(base0, g + 2, jn)
                        gather_start(jn, b)
                return carry

            lax.fori_loop(0, spw0 // UNROLL, step, 0)

        @pl.when(c == 1)
        def _():
            def step(i, carry):
                g0 = i * NI
                for j in range(NI):
                    g = g0 + j
                    idx_wait(base1, g, j)
                    gather_start(j, 0)
                    gather_wait(j, 0)
                    pltpu.sync_copy(rows[0], agg_sh.at[dsts[j]], add=True)

                    @pl.when(g + NI < spw1)
                    def _():
                        idx_start(base1, g + NI, j)
                return carry

            lax.fori_loop(0, spw1 // NI, step, 0)

        plsc.subcore_barrier()
        pltpu.sync_copy(agg_sh.at[rslc], out_hbm.at[c, rslc])

    return sc_agg


def _dense_body(n, p0r, p1r, w1r, b1r, w2r, b2r, gr, br, outr):
    h = p0r[...][:n] + p1r[...][:n]
    a = jnp.dot(h, w1r[...], preferred_element_type=jnp.float32) + b1r[...]
    a = jnp.maximum(a, 0.0)
    h2 = jnp.dot(a, w2r[...], preferred_element_type=jnp.float32) + b2r[...]
    mean = jnp.mean(h2, axis=0, keepdims=True)
    cent = h2 - mean
    var = jnp.mean(cent * cent, axis=0, keepdims=True)
    scale = lax.rsqrt(var + 1e-5) * gr[...]
    outr[...] = jnp.maximum(cent * scale + br[...], 0.0)


def kernel(x, edge_index, W1, b1, W2, b2, gamma, beta):
    n, d = x.shape
    e = edge_index.shape[1]
    steps = -(-e // (NS * K))
    spw0 = int(steps * SHARE0)
    spw0 = -(-spw0 // UNROLL) * UNROLL
    spw1 = -(-(steps - spw0) // NI) * NI
    spw1 = max(spw1, NI)
    e_pad = (spw0 + spw1) * K * NS
    n_pad = -(-(n + 1) // (NS * 8)) * (NS * 8)
    dummy = n_pad - 1

    src = edge_index[0].astype(jnp.int32)
    dst = edge_index[1].astype(jnp.int32)
    src_p = jnp.concatenate([src, jnp.zeros((e_pad - e,), jnp.int32)])
    dst_p = jnp.concatenate([dst, jnp.full((e_pad - e,), dummy, jnp.int32)])
    xp = jnp.pad(x, ((0, n_pad - n), (0, 0)))
    zz = jnp.zeros((n_pad, d), jnp.float32)

    slabs = _sc_agg_call(n_pad, spw0, spw1, d)(xp, zz, src_p, dst_p)

    out = pl.pallas_call(
        functools.partial(_dense_body, n),
        out_shape=jax.ShapeDtypeStruct((n, d), jnp.float32),
    )(slabs[0], slabs[1], W1.T, b1.reshape(1, d), W2.T,
      b2.reshape(1, d), gamma.reshape(1, d), beta.reshape(1, d))
    return out


# speedup vs baseline: 5.2202x; 1.5330x over previous
import functools

import jax
import jax.numpy as jnp
from jax import lax
from jax.experimental import pallas as pl
from jax.experimental.pallas import tpu as pltpu
from jax.experimental.pallas import tpu_sc as plsc

NC = 2
NS = 16
K = 128
NI = 3
NB = 2
UNROLL = 6
SHARE0 = 0.80


def _sc_agg_call(n_pad, spw0, spw1, d):
    mesh = plsc.VectorSubcoreMesh(core_axis_name="c", subcore_axis_name="s")
    rows_per_tile = n_pad // NS

    @functools.partial(
        pl.kernel,
        mesh=mesh,
        out_type=jax.ShapeDtypeStruct((NC, n_pad, d), jnp.float32),
        scratch_types=(
            [pltpu.VMEM((K,), jnp.int32) for _ in range(2 * NI)]
            + [pltpu.VMEM((K, d), jnp.float32) for _ in range(NB)]
            + [pltpu.VMEM_SHARED((n_pad, d), jnp.float32)]
            + [pltpu.SemaphoreType.DMA for _ in range(NI + NB + 1)]
        ),
    )
    def sc_agg(xp_hbm, zz_hbm, src_hbm, dst_hbm, out_hbm,
               src0, src1, src2, dst0, dst1, dst2, rows0, rows1,
               agg_sh, isem0, isem1, isem2, gsem0, gsem1, ssem):
        c = lax.axis_index("c")
        s = lax.axis_index("s")
        srcs = (src0, src1, src2)
        dsts = (dst0, dst1, dst2)
        isems = (isem0, isem1, isem2)
        rows = (rows0, rows1)
        gsems = (gsem0, gsem1)
        rslc = pl.ds(s * rows_per_tile, rows_per_tile)
        base0 = s * (spw0 * K)
        base1 = (NS * spw0 + s * spw1) * K

        def idx_start(base, g, j):
            off = pl.multiple_of(base + g * K, K)
            pltpu.async_copy(src_hbm.at[pl.ds(off, K)], srcs[j], isems[j])
            pltpu.async_copy(dst_hbm.at[pl.ds(off, K)], dsts[j], isems[j])

        def idx_wait(base, g, j):
            off = pl.multiple_of(base + g * K, K)
            pltpu.make_async_copy(src_hbm.at[pl.ds(off, K)], srcs[j],
                                  isems[j]).wait()
            pltpu.make_async_copy(dst_hbm.at[pl.ds(off, K)], dsts[j],
                                  isems[j]).wait()

        def gather_start(j, b):
            pltpu.async_copy(xp_hbm.at[srcs[j]], rows[b], gsems[b])

        def gather_wait(j, b):
            pltpu.make_async_copy(xp_hbm.at[srcs[j]], rows[b],
                                  gsems[b]).wait()

        @pl.when(c == 0)
        def _():
            for g in range(NI):
                idx_start(base0, g, g)
            pltpu.async_copy(xp_hbm.at[rslc], agg_sh.at[rslc], ssem)
            pltpu.make_async_copy(xp_hbm.at[rslc], agg_sh.at[rslc],
                                  ssem).wait()

        @pl.when(c == 1)
        def _():
            pltpu.async_copy(zz_hbm.at[rslc], agg_sh.at[rslc], ssem)
            pltpu.make_async_copy(zz_hbm.at[rslc], agg_sh.at[rslc],
                                  ssem).wait()

        plsc.subcore_barrier()

        @pl.when(c == 0)
        def _():
            idx_wait(base0, 0, 0)
            gather_start(0, 0)
            idx_wait(base0, 1, 1)
            gather_start(1, 1)

            def step(i, carry):
                g0 = i * UNROLL
                for u in range(UNROLL):
                    g = g0 + u
                    b = u % NB
                    j = u % NI
                    jn = (u + 2) % NI
                    gather_wait(j, b)
                    pltpu.sync_copy(rows[b], agg_sh.at[dsts[j]], add=True)

                    @pl.when(g + NI < spw0)
                    def _():
                        idx_start(base0, g + NI, j)

                    @pl.when(g + 2 < spw0)
                    def _():
                        idx_wait(base0, g + 2, jn)
                        gather_start(jn, b)
                return carry

            lax.fori_loop(0, spw0 // UNROLL, step, 0)

        @pl.when(c == 1)
        def _():
            def step(g, carry):
                off = pl.multiple_of(base1 + g * K, K)
                pltpu.sync_copy(src_hbm.at[pl.ds(off, K)], srcs[0])
                pltpu.sync_copy(dst_hbm.at[pl.ds(off, K)], dsts[0])
                gather_start(0, 0)
                gather_wait(0, 0)
                pltpu.sync_copy(rows[0], agg_sh.at[dsts[0]], add=True)
                return carry

            lax.fori_loop(0, spw1, step, 0)

        plsc.subcore_barrier()
        pltpu.sync_copy(agg_sh.at[rslc], out_hbm.at[c, rslc])

    return sc_agg


def _dense_body(n, p0r, p1r, w1r, b1r, w2r, b2r, gr, br, outr):
    h = p0r[...][:n] + p1r[...][:n]
    a = jnp.dot(h, w1r[...], preferred_element_type=jnp.float32) + b1r[...]
    a = jnp.maximum(a, 0.0)
    h2 = jnp.dot(a, w2r[...], preferred_element_type=jnp.float32) + b2r[...]
    mean = jnp.mean(h2, axis=0, keepdims=True)
    cent = h2 - mean
    var = jnp.mean(cent * cent, axis=0, keepdims=True)
    scale = lax.rsqrt(var + 1e-5) * gr[...]
    outr[...] = jnp.maximum(cent * scale + br[...], 0.0)


def kernel(x, edge_index, W1, b1, W2, b2, gamma, beta):
    n, d = x.shape
    e = edge_index.shape[1]
    steps = -(-e // (NS * K))
    spw0 = int(steps * SHARE0)
    spw0 = -(-spw0 // UNROLL) * UNROLL
    spw1 = max(steps - spw0, 1)
    e_pad = (spw0 + spw1) * K * NS
    n_pad = -(-(n + 1) // (NS * 8)) * (NS * 8)
    dummy = n_pad - 1

    src = edge_index[0].astype(jnp.int32)
    dst = edge_index[1].astype(jnp.int32)
    src_p = jnp.concatenate([src, jnp.zeros((e_pad - e,), jnp.int32)])
    dst_p = jnp.concatenate([dst, jnp.full((e_pad - e,), dummy, jnp.int32)])
    xp = jnp.pad(x, ((0, n_pad - n), (0, 0)))
    zz = jnp.zeros((n_pad, d), jnp.float32)

    slabs = _sc_agg_call(n_pad, spw0, spw1, d)(xp, zz, src_p, dst_p)

    out = pl.pallas_call(
        functools.partial(_dense_body, n),
        out_shape=jax.ShapeDtypeStruct((n, d), jnp.float32),
    )(slabs[0], slabs[1], W1.T, b1.reshape(1, d), W2.T,
      b2.reshape(1, d), gamma.reshape(1, d), beta.reshape(1, d))
    return out


# speedup vs baseline: 6.2664x; 1.2004x over previous
import functools

import jax
import jax.numpy as jnp
from jax import lax
from jax.experimental import pallas as pl
from jax.experimental.pallas import tpu as pltpu
from jax.experimental.pallas import tpu_sc as plsc

NC = 2
NS = 16
K = 128
NI = 3
NB = 2
UNROLL = 6
SHARE0 = 0.845


def _split_rows(n_rows):
    per = -(-n_rows // NS)
    per = -(-per // 8) * 8
    slices = []
    off = 0
    for s in range(NS):
        size = min(per, n_rows - off)
        slices.append((off, max(size, 0)))
        off += size
    return slices


def _sc_agg_call(n, n_pad, d, e_off, spw0, q1, r1):
    mesh = plsc.VectorSubcoreMesh(core_axis_name="c", subcore_axis_name="s")
    rows_per_tile = n_pad // NS
    xslices = _split_rows(n)
    start1 = NS * spw0

    @functools.partial(
        pl.kernel,
        mesh=mesh,
        out_type=jax.ShapeDtypeStruct((NC, n_pad, d), jnp.float32),
        scratch_types=(
            [pltpu.VMEM((K,), jnp.int32) for _ in range(2 * NI)]
            + [pltpu.VMEM((K, d), jnp.float32) for _ in range(NB)]
            + [pltpu.VMEM_SHARED((n_pad, d), jnp.float32)]
            + [pltpu.SemaphoreType.DMA for _ in range(NI + NB + 1)]
        ),
    )
    def sc_agg(x_hbm, zz_hbm, ei_hbm, out_hbm,
               src0, src1, src2, dst0, dst1, dst2, rows0, rows1,
               agg_sh, isem0, isem1, isem2, gsem0, gsem1, ssem):
        c = lax.axis_index("c")
        s = lax.axis_index("s")
        srcs = (src0, src1, src2)
        dsts = (dst0, dst1, dst2)
        isems = (isem0, isem1, isem2)
        rows = (rows0, rows1)
        gsems = (gsem0, gsem1)
        rslc = pl.ds(s * rows_per_tile, rows_per_tile)
        base0 = s * (spw0 * K)
        cnt1 = q1 + jnp.where(s < r1, 1, 0)
        base1 = (start1 + q1 * s + jnp.minimum(s, r1)) * K

        def idx_start(base, g, j):
            off = pl.multiple_of(base + g * K, K)
            pltpu.async_copy(ei_hbm.at[pl.ds(off, K)], srcs[j], isems[j])
            pltpu.async_copy(ei_hbm.at[pl.ds(e_off + off, K)], dsts[j],
                             isems[j])

        def idx_wait(base, g, j):
            off = pl.multiple_of(base + g * K, K)
            pltpu.make_async_copy(ei_hbm.at[pl.ds(off, K)], srcs[j],
                                  isems[j]).wait()
            pltpu.make_async_copy(ei_hbm.at[pl.ds(e_off + off, K)], dsts[j],
                                  isems[j]).wait()

        def gather_start(j, b):
            pltpu.async_copy(x_hbm.at[srcs[j]], rows[b], gsems[b])

        def gather_wait(j, b):
            pltpu.make_async_copy(x_hbm.at[srcs[j]], rows[b],
                                  gsems[b]).wait()

        @pl.when(c == 0)
        def _():
            for g in range(NI):
                idx_start(base0, g, g)
            for t, (xo, xs_) in enumerate(xslices):
                if xs_ > 0:
                    @pl.when(s == t)
                    def _():
                        pltpu.async_copy(x_hbm.at[pl.ds(xo, xs_)],
                                         agg_sh.at[pl.ds(xo, xs_)], ssem)
                        pltpu.make_async_copy(
                            x_hbm.at[pl.ds(xo, xs_)],
                            agg_sh.at[pl.ds(xo, xs_)], ssem).wait()

        @pl.when(c == 1)
        def _():
            pltpu.async_copy(zz_hbm.at[rslc], agg_sh.at[rslc], ssem)
            pltpu.make_async_copy(zz_hbm.at[rslc], agg_sh.at[rslc],
                                  ssem).wait()

        plsc.subcore_barrier()

        @pl.when(c == 0)
        def _():
            idx_wait(base0, 0, 0)
            gather_start(0, 0)
            idx_wait(base0, 1, 1)
            gather_start(1, 1)

            def step(i, carry):
                g0 = i * UNROLL
                for u in range(UNROLL):
                    g = g0 + u
                    b = u % NB
                    j = u % NI
                    jn = (u + 2) % NI
                    gather_wait(j, b)
                    pltpu.sync_copy(rows[b], agg_sh.at[dsts[j]], add=True)

                    @pl.when(g + NI < spw0)
                    def _():
                        idx_start(base0, g + NI, j)

                    @pl.when(g + 2 < spw0)
                    def _():
                        idx_wait(base0, g + 2, jn)
                        gather_start(jn, b)
                return carry

            lax.fori_loop(0, spw0 // UNROLL, step, 0)

        @pl.when(c == 1)
        def _():
            def step(g, carry):
                off = pl.multiple_of(base1 + g * K, K)
                pltpu.sync_copy(ei_hbm.at[pl.ds(off, K)], srcs[0])
                pltpu.sync_copy(ei_hbm.at[pl.ds(e_off + off, K)], dsts[0])
                gather_start(0, 0)
                gather_wait(0, 0)
                pltpu.sync_copy(rows[0], agg_sh.at[dsts[0]], add=True)
                return carry

            lax.fori_loop(0, cnt1, step, 0)

        plsc.subcore_barrier()
        pltpu.sync_copy(agg_sh.at[rslc], out_hbm.at[c, rslc])

    return sc_agg


def _dense_body(n, sr, w1r, b1r, w2r, b2r, gr, br, outr):
    h = sr[0, :n, :] + sr[1, :n, :]
    a = jnp.dot(h, w1r[...], preferred_element_type=jnp.float32) + b1r[...]
    a = jnp.maximum(a, 0.0)
    h2 = jnp.dot(a, w2r[...], preferred_element_type=jnp.float32) + b2r[...]
    mean = jnp.mean(h2, axis=0, keepdims=True)
    cent = h2 - mean
    var = jnp.mean(cent * cent, axis=0, keepdims=True)
    scale = lax.rsqrt(var + 1e-5) * gr[...]
    outr[...] = jnp.maximum(cent * scale + br[...], 0.0)


def kernel(x, edge_index, W1, b1, W2, b2, gamma, beta):
    n, d = x.shape
    e = edge_index.shape[1]
    n_pad = -(-n // (NS * 8)) * (NS * 8)

    steps = e // K
    assert e % K == 0 and e % 8 == 0, "edge count must be step-aligned"
    spw0 = int(steps * SHARE0) // NS
    spw0 = (spw0 // UNROLL) * UNROLL
    q1, r1 = divmod(steps - NS * spw0, NS)

    ei = edge_index.astype(jnp.int32).reshape(2 * e)
    zz = jnp.zeros((n_pad, d), jnp.float32)

    slabs = _sc_agg_call(n, n_pad, d, e, spw0, q1, r1)(x, zz, ei)

    out = pl.pallas_call(
        functools.partial(_dense_body, n),
        out_shape=jax.ShapeDtypeStruct((n, d), jnp.float32),
    )(slabs, W1.T, b1.reshape(1, d), W2.T,
      b2.reshape(1, d), gamma.reshape(1, d), beta.reshape(1, d))
    return out


# speedup vs baseline: 6.6978x; 1.0688x over previous
import functools

import jax
import jax.numpy as jnp
from jax import lax
from jax.experimental import pallas as pl
from jax.experimental.pallas import tpu as pltpu
from jax.experimental.pallas import tpu_sc as plsc

NC = 2
NS = 16
K = 128
NI = 3
NB = 2
UNROLL = 6
SHARE0 = 0.77


def _split_rows(n_rows):
    per = -(-n_rows // NS)
    per = -(-per // 8) * 8
    slices = []
    off = 0
    for s in range(NS):
        size = min(per, n_rows - off)
        slices.append((off, max(size, 0)))
        off += size
    return slices


def _sc_agg_call(n, n_pad, d, e_off, spw0, q1, r1):
    mesh = plsc.VectorSubcoreMesh(core_axis_name="c", subcore_axis_name="s")
    rows_per_tile = n_pad // NS
    xslices = _split_rows(n)
    start1 = NS * spw0

    @functools.partial(
        pl.kernel,
        mesh=mesh,
        out_type=jax.ShapeDtypeStruct((NC, n_pad, d), jnp.float32),
        scratch_types=(
            [pltpu.VMEM((K,), jnp.int32) for _ in range(2 * NI)]
            + [pltpu.VMEM((K, d), jnp.float32) for _ in range(NB)]
            + [pltpu.VMEM_SHARED((n_pad, d), jnp.float32)]
            + [pltpu.SemaphoreType.DMA for _ in range(NI + NB + 1)]
        ),
    )
    def sc_agg(x_hbm, zz_hbm, ei_hbm, out_hbm,
               src0, src1, src2, dst0, dst1, dst2, rows0, rows1,
               agg_sh, isem0, isem1, isem2, gsem0, gsem1, ssem):
        c = lax.axis_index("c")
        s = lax.axis_index("s")
        srcs = (src0, src1, src2)
        dsts = (dst0, dst1, dst2)
        isems = (isem0, isem1, isem2)
        rows = (rows0, rows1)
        gsems = (gsem0, gsem1)
        rslc = pl.ds(s * rows_per_tile, rows_per_tile)
        base0 = s * (spw0 * K)
        cnt1 = q1 + jnp.where(s < r1, 1, 0)
        base1 = (start1 + q1 * s + jnp.minimum(s, r1)) * K

        def idx_start(base, g, j):
            off = pl.multiple_of(base + g * K, K)
            pltpu.async_copy(ei_hbm.at[pl.ds(off, K)], srcs[j], isems[j])
            pltpu.async_copy(ei_hbm.at[pl.ds(e_off + off, K)], dsts[j],
                             isems[j])

        def idx_wait(base, g, j):
            off = pl.multiple_of(base + g * K, K)
            pltpu.make_async_copy(ei_hbm.at[pl.ds(off, K)], srcs[j],
                                  isems[j]).wait()
            pltpu.make_async_copy(ei_hbm.at[pl.ds(e_off + off, K)], dsts[j],
                                  isems[j]).wait()

        def gather_start(j, b):
            pltpu.async_copy(x_hbm.at[srcs[j]], rows[b], gsems[b])

        def gather_wait(j, b):
            pltpu.make_async_copy(x_hbm.at[srcs[j]], rows[b],
                                  gsems[b]).wait()

        @pl.when(c == 0)
        def _():
            for g in range(NI):
                idx_start(base0, g, g)
            for t, (xo, xs_) in enumerate(xslices):
                if xs_ > 0:
                    @pl.when(s == t)
                    def _():
                        pltpu.async_copy(x_hbm.at[pl.ds(xo, xs_)],
                                         agg_sh.at[pl.ds(xo, xs_)], ssem)
                        pltpu.make_async_copy(
                            x_hbm.at[pl.ds(xo, xs_)],
                            agg_sh.at[pl.ds(xo, xs_)], ssem).wait()

        @pl.when(c == 1)
        def _():
            pltpu.async_copy(zz_hbm.at[rslc], agg_sh.at[rslc], ssem)
            pltpu.make_async_copy(zz_hbm.at[rslc], agg_sh.at[rslc],
                                  ssem).wait()

        plsc.subcore_barrier()

        @pl.when(c == 0)
        def _():
            idx_wait(base0, 0, 0)
            gather_start(0, 0)
            idx_wait(base0, 1, 1)
            gather_start(1, 1)

            def step(i, carry):
                g0 = i * UNROLL
                for u in range(UNROLL):
                    g = g0 + u
                    b = u % NB
                    j = u % NI
                    jn = (u + 2) % NI
                    gather_wait(j, b)
                    pltpu.sync_copy(rows[b], agg_sh.at[dsts[j]], add=True)

                    @pl.when(g + NI < spw0)
                    def _():
                        idx_start(base0, g + NI, j)

                    @pl.when(g + 2 < spw0)
                    def _():
                        idx_wait(base0, g + 2, jn)
                        gather_start(jn, b)
                return carry

            lax.fori_loop(0, spw0 // UNROLL, step, 0)

        @pl.when(c == 1)
        def _():
            def step(g, carry):
                off = pl.multiple_of(base1 + g * K, K)
                pltpu.sync_copy(ei_hbm.at[pl.ds(off, K)], srcs[0])
                pltpu.sync_copy(ei_hbm.at[pl.ds(e_off + off, K)], dsts[0])
                gather_start(0, 0)
                gather_wait(0, 0)
                pltpu.sync_copy(rows[0], agg_sh.at[dsts[0]], add=True)
                return carry

            lax.fori_loop(0, cnt1, step, 0)

        plsc.subcore_barrier()
        pltpu.sync_copy(agg_sh.at[rslc], out_hbm.at[c, rslc])

    return sc_agg


def _dense_body(n, sr, w1r, b1r, w2r, b2r, gr, br, outr):
    h = sr[0, :n, :] + sr[1, :n, :]
    a = jnp.dot(h, w1r[...], preferred_element_type=jnp.float32) + b1r[...]
    a = jnp.maximum(a, 0.0)
    h2 = jnp.dot(a, w2r[...], preferred_element_type=jnp.float32) + b2r[...]
    mean = jnp.mean(h2, axis=0, keepdims=True)
    cent = h2 - mean
    var = jnp.mean(cent * cent, axis=0, keepdims=True)
    scale = lax.rsqrt(var + 1e-5) * gr[...]
    outr[...] = jnp.maximum(cent * scale + br[...], 0.0)


def kernel(x, edge_index, W1, b1, W2, b2, gamma, beta):
    n, d = x.shape
    e = edge_index.shape[1]
    n_pad = -(-n // (NS * 8)) * (NS * 8)

    steps = e // K
    assert e % K == 0 and e % 8 == 0, "edge count must be step-aligned"
    spw0 = int(steps * SHARE0) // NS
    spw0 = (spw0 // UNROLL) * UNROLL
    q1, r1 = divmod(steps - NS * spw0, NS)

    ei = edge_index.astype(jnp.int32).reshape(2 * e)
    zz = jnp.zeros((n_pad, d), jnp.float32)

    slabs = _sc_agg_call(n, n_pad, d, e, spw0, q1, r1)(x, zz, ei)

    out = pl.pallas_call(
        functools.partial(_dense_body, n),
        out_shape=jax.ShapeDtypeStruct((n, d), jnp.float32),
    )(slabs, W1.T, b1.reshape(1, d), W2.T,
      b2.reshape(1, d), gamma.reshape(1, d), beta.reshape(1, d))
    return out
